# Initial kernel scaffold; baseline (speedup 1.0000x reference)
import jax
import jax.numpy as jnp
from jax.experimental import pallas as pl


def kernel(input_ids, table):
    raise NotImplementedError("write your pallas kernel here")


# speedup vs baseline: 1.0936x; 1.0936x over previous
import functools

import jax
import jax.numpy as jnp
from jax import lax
from jax.experimental import pallas as pl
from jax.experimental.pallas import tpu as pltpu
from jax.experimental.pallas import tpu_sc as plsc

NUM_EMBEDDINGS = 1000000
EMBEDDING_DIM = 32
BATCH = 16384
HIST_LEN = 50

_INFO = plsc.get_sparse_core_info()
_NC = _INFO.num_cores
_NS = _INFO.num_subcores
_NW = _NC * _NS

_B_TOTAL = BATCH * HIST_LEN
_IDX_MINOR = 128
_G = 8
_ROWS_PER_STEP = _G * _IDX_MINOR
_B_PER_W = _B_TOTAL // _NW
_N_STEPS = _B_PER_W // _ROWS_PER_STEP
_IDX_ROWS_PER_W = _B_PER_W // _IDX_MINOR

assert _B_PER_W % _ROWS_PER_STEP == 0


def _sc_gather(ids2d, table):
    mesh = plsc.VectorSubcoreMesh(core_axis_name="c", subcore_axis_name="s")

    @functools.partial(
        pl.kernel,
        mesh=mesh,
        out_type=jax.ShapeDtypeStruct((_B_TOTAL, EMBEDDING_DIM), jnp.float32),
        scratch_types=[
            pltpu.VMEM((_G, _IDX_MINOR), jnp.int32),
            pltpu.VMEM((_ROWS_PER_STEP, EMBEDDING_DIM), jnp.float32),
            pltpu.SemaphoreType.DMA,
        ],
        compiler_params=pltpu.CompilerParams(use_tc_tiling_on_sc=False),
    )
    def k(ids_hbm, table_hbm, out_hbm, idx_v, rows_v, sem):
        wid = lax.axis_index("s") * _NC + lax.axis_index("c")
        idx_row0 = wid * _IDX_ROWS_PER_W
        out_row0 = wid * _B_PER_W

        def step(o, carry):
            pltpu.sync_copy(ids_hbm.at[pl.ds(idx_row0 + o * _G, _G)], idx_v)
            handles = []
            for j in range(_G):
                handles.append(
                    pltpu.async_copy(
                        table_hbm.at[idx_v.at[j]],
                        rows_v.at[pl.ds(j * _IDX_MINOR, _IDX_MINOR)],
                        sem,
                    )
                )
            for h in handles:
                h.wait()
            pltpu.sync_copy(
                rows_v,
                out_hbm.at[pl.ds(out_row0 + o * _ROWS_PER_STEP, _ROWS_PER_STEP)],
            )
            return carry

        lax.fori_loop(0, _N_STEPS, step, 0)

    return k(ids2d, table)


def kernel(input_ids, table):
    ids2d = input_ids.reshape(_B_TOTAL // _IDX_MINOR, _IDX_MINOR).astype(jnp.int32)
    out = _sc_gather(ids2d, table)
    return out.reshape(BATCH, HIST_LEN, EMBEDDING_DIM)


# speedup vs baseline: 1.1124x; 1.0172x over previous
import functools

import jax
import jax.numpy as jnp
from jax import lax
from jax.experimental import pallas as pl
from jax.experimental.pallas import tpu as pltpu
from jax.experimental.pallas import tpu_sc as plsc

NUM_EMBEDDINGS = 1000000
EMBEDDING_DIM = 32
BATCH = 16384
HIST_LEN = 50

_INFO = plsc.get_sparse_core_info()
_NC = _INFO.num_cores
_NS = _INFO.num_subcores
_NW = _NC * _NS

_B_TOTAL = BATCH * HIST_LEN
_IDX_MINOR = 128
_G = 5
_CH = _G * _IDX_MINOR
_RING = 4
_B_PER_W = _B_TOTAL // _NW
_N_CHUNKS = _B_PER_W // _CH
_IDX_ROWS_PER_W = _B_PER_W // _IDX_MINOR

assert _B_PER_W % _CH == 0 and _N_CHUNKS % _RING == 0


def _sc_gather(ids2d, table):
    mesh = plsc.VectorSubcoreMesh(core_axis_name="c", subcore_axis_name="s")

    @functools.partial(
        pl.kernel,
        mesh=mesh,
        out_type=jax.ShapeDtypeStruct((_B_TOTAL, EMBEDDING_DIM), jnp.float32),
        scratch_types=[
            pltpu.VMEM((_IDX_ROWS_PER_W, _IDX_MINOR), jnp.int32),
            pltpu.VMEM((_RING * _CH, EMBEDDING_DIM), jnp.float32),
            pltpu.SemaphoreType.DMA,
            pltpu.SemaphoreType.DMA,
            pltpu.SemaphoreType.DMA,
            pltpu.SemaphoreType.DMA,
        ],
        compiler_params=pltpu.CompilerParams(use_tc_tiling_on_sc=False),
    )
    def k(ids_hbm, table_hbm, out_hbm, idx_v, rows_v, s0, s1, s2, s3):
        sems = (s0, s1, s2, s3)
        wid = lax.axis_index("s") * _NC + lax.axis_index("c")
        out_row0 = wid * _B_PER_W

        pltpu.sync_copy(ids_hbm.at[pl.ds(wid * _IDX_ROWS_PER_W, _IDX_ROWS_PER_W)], idx_v)

        def fire(c, r):
            for j in range(_G):
                pltpu.async_copy(
                    table_hbm.at[idx_v.at[c * _G + j]],
                    rows_v.at[pl.ds(r * _CH + j * _IDX_MINOR, _IDX_MINOR)],
                    sems[r],
                )

        def drain(r):
            pltpu.make_async_copy(
                table_hbm.at[pl.ds(0, _CH)],
                rows_v.at[pl.ds(r * _CH, _CH)],
                sems[r],
            ).wait()

        def store(c, r):
            pltpu.sync_copy(
                rows_v.at[pl.ds(r * _CH, _CH)],
                out_hbm.at[pl.ds(out_row0 + c * _CH, _CH)],
            )

        for r in range(_RING):
            fire(r, r)

        def body(i, carry):
            for r in range(_RING):
                c = i * _RING + r
                drain(r)
                store(c, r)
                fire(c + _RING, r)
            return carry

        lax.fori_loop(0, _N_CHUNKS // _RING - 1, body, 0)

        for r in range(_RING):
            drain(r)
            store(_N_CHUNKS - _RING + r, r)

    return k(ids2d, table)


def kernel(input_ids, table):
    ids2d = input_ids.reshape(_B_TOTAL // _IDX_MINOR, _IDX_MINOR).astype(jnp.int32)
    out = _sc_gather(ids2d, table)
    return out.reshape(BATCH, HIST_LEN, EMBEDDING_DIM)


# speedup vs baseline: 1.8039x; 1.6216x over previous
import functools

import jax
import jax.numpy as jnp
from jax import lax
from jax.experimental import pallas as pl
from jax.experimental.pallas import tpu as pltpu
from jax.experimental.pallas import tpu_sc as plsc

NUM_EMBEDDINGS = 1000000
EMBEDDING_DIM = 32
BATCH = 16384
HIST_LEN = 50

_INFO = plsc.get_sparse_core_info()
_NC = _INFO.num_cores
_NS = _INFO.num_subcores
_NW = _NC * _NS

_B_PER_W = BATCH // _NW
_G = 4
_RING = 4
_N_CHUNKS = _B_PER_W // _G

assert _B_PER_W % _G == 0 and _N_CHUNKS % _RING == 0


def _sc_gather(ids, table):
    mesh = plsc.VectorSubcoreMesh(core_axis_name="c", subcore_axis_name="s")

    @functools.partial(
        pl.kernel,
        mesh=mesh,
        out_type=jax.ShapeDtypeStruct((BATCH, HIST_LEN, EMBEDDING_DIM), jnp.float32),
        scratch_types=[
            pltpu.VMEM((_B_PER_W, HIST_LEN), jnp.int32),
            pltpu.VMEM((_RING, _G, HIST_LEN, EMBEDDING_DIM), jnp.float32),
            pltpu.SemaphoreType.DMA,
            pltpu.SemaphoreType.DMA,
            pltpu.SemaphoreType.DMA,
            pltpu.SemaphoreType.DMA,
        ],
        compiler_params=pltpu.CompilerParams(use_tc_tiling_on_sc=False),
    )
    def k(ids_hbm, table_hbm, out_hbm, idx_v, rows_v, s0, s1, s2, s3):
        sems = (s0, s1, s2, s3)
        wid = lax.axis_index("s") * _NC + lax.axis_index("c")
        b0 = wid * _B_PER_W

        pltpu.sync_copy(ids_hbm.at[pl.ds(b0, _B_PER_W)], idx_v)

        def fire(c, r):
            for j in range(_G):
                pltpu.async_copy(
                    table_hbm.at[idx_v.at[c * _G + j]],
                    rows_v.at[r].at[j],
                    sems[r],
                )

        def drain(r):
            pltpu.make_async_copy(
                out_hbm.at[pl.ds(0, _G)],
                rows_v.at[r],
                sems[r],
            ).wait()

        def store(c, r):
            pltpu.sync_copy(
                rows_v.at[r],
                out_hbm.at[pl.ds(b0 + c * _G, _G)],
            )

        for r in range(_RING):
            fire(r, r)

        def body(i, carry):
            for r in range(_RING):
                c = i * _RING + r
                drain(r)
                store(c, r)
                fire(c + _RING, r)
            return carry

        lax.fori_loop(0, _N_CHUNKS // _RING - 1, body, 0)

        for r in range(_RING):
            drain(r)
            store(_N_CHUNKS - _RING + r, r)

    return k(ids, table)


def kernel(input_ids, table):
    return _sc_gather(input_ids.astype(jnp.int32), table)
